# Initial kernel scaffold; baseline (speedup 1.0000x reference)
import jax
import jax.numpy as jnp
from jax.experimental import pallas as pl


def kernel(x, edge_index, batch, target, W1, b1, W2, b2, W3, b3, Wg1, bg1, Wg2, bg2, emb, Wc, bc, Wxt, bxt, Wf1, bf1, Wf2, bf2, Wo, bo):
    raise NotImplementedError("write your pallas kernel here")


# speedup vs baseline: 18.5848x; 18.5848x over previous
import functools

import jax
import jax.numpy as jnp
from jax import lax
from jax.experimental import pallas as pl
from jax.experimental.pallas import tpu as pltpu
from jax.experimental.pallas import tpu_sc as plsc

N = 10000
E = 320000
B = 64
L = 1000
D = 128

NC = 2
NS = 16
NW = NC * NS
CH = 128

NPAD = 10240
NCHUNKS = E // CH
NJUNK = 4
CBIG, NBIG = 80, 25
CSLAB = CBIG
NROWS_IDX = 2512
HCHUNKS = NCHUNKS // 2
HJUNK = 6
HBIG, HNBIG = 40, 29
HSLAB = HBIG
HROWS_IDX = 1264
ROWS_PT = NPAD // NW
NACC = N
ARPT = 632

SEGPAD = 72


def _sc_mesh():
  return plsc.VectorSubcoreMesh(
      core_axis_name="c", subcore_axis_name="s", num_cores=NC,
      num_subcores=NS)


def _tile_chunks(wid):
  base = jnp.where(wid < NBIG, CBIG * wid,
                   CBIG * NBIG + (CBIG - 8) * (wid - NBIG))
  nch = jnp.where(wid < NBIG, CBIG, CBIG - 8)
  return base, nch


def _tile_chunks_half(wid):
  base = jnp.where(wid < HNBIG, HBIG * wid,
                   HBIG * HNBIG + (HBIG - 8) * (wid - HNBIG))
  nch = jnp.where(wid < HNBIG, HBIG, HBIG - 8)
  return base, nch


def _acc_pieces(sid):
  last = sid == NS - 1
  n15 = NACC - (NS - 1) * ARPT
  pieces_main = [(o, min(CH, ARPT - o)) for o in range(0, ARPT, CH)]
  pieces_last = [(o, min(CH, n15 - o)) for o in range(0, n15, CH)]
  return last, pieces_main, pieces_last


def _zero_acc(zeros_hbm, shared, sid):
  last, pm, plast = _acc_pieces(sid)

  @pl.when(jnp.logical_not(last))
  def _():
    for o, sz in pm:
      pltpu.sync_copy(zeros_hbm.at[pl.ds(0, sz)],
                      shared.at[pl.ds(sid * ARPT + o, sz)])

  @pl.when(last)
  def _():
    for o, sz in plast:
      pltpu.sync_copy(zeros_hbm.at[pl.ds(0, sz)],
                      shared.at[pl.ds(sid * ARPT + o, sz)])


def _read_acc(shared, out_hbm, cid, sid):
  last, pm, plast = _acc_pieces(sid)

  @pl.when(jnp.logical_not(last))
  def _():
    for o, sz in pm:
      pltpu.sync_copy(shared.at[pl.ds(sid * ARPT + o, sz)],
                      out_hbm.at[cid, pl.ds(sid * ARPT + o, sz)])

  @pl.when(last)
  def _():
    for o, sz in plast:
      pltpu.sync_copy(shared.at[pl.ds(sid * ARPT + o, sz)],
                      out_hbm.at[cid, pl.ds(sid * ARPT + o, sz)])


def _deg_body(dstp_hbm, ones_hbm, zeros_hbm, out_hbm,
              idx_v, ones_v, sem, shared):
  cid = lax.axis_index("c")
  sid = lax.axis_index("s")
  wid = sid * NC + cid
  base, nch = _tile_chunks(wid)
  pltpu.sync_copy(dstp_hbm.at[pl.ds(base, CSLAB)], idx_v)
  pltpu.sync_copy(ones_hbm.at[pl.ds(0, CH)], ones_v)
  _zero_acc(zeros_hbm, shared, sid)
  plsc.subcore_barrier()

  def step(j, _):
    pltpu.async_copy(ones_v, shared.at[idx_v.at[j]], sem, add=True).wait()
    return _

  lax.fori_loop(0, nch, step, None)
  plsc.subcore_barrier()
  _read_acc(shared, out_hbm, cid, sid)


def _deg_call(dstp, ones16, zeros16):
  return pl.kernel(
      _deg_body,
      out_type=jax.ShapeDtypeStruct((NC, NPAD, D), jnp.float32),
      mesh=_sc_mesh(),
      scratch_types=[
          pltpu.VMEM((CSLAB, CH), jnp.int32),
          pltpu.VMEM((CH, D), jnp.float32),
          pltpu.SemaphoreType.DMA,
          pltpu.VMEM_SHARED((NACC, D), jnp.float32),
      ],
  )(dstp, ones16, zeros16)


def _agg_body(table_hbm, srcp_hbm, dstp_hbm, zeros_hbm, out_hbm,
              sidx_v, didx_v, rows_v, gsem, ssem, shared):
  cid = lax.axis_index("c")
  sid = lax.axis_index("s")
  wid = sid * NC + cid
  base, nch = _tile_chunks_half(wid)
  pltpu.sync_copy(srcp_hbm.at[pl.ds(base, HSLAB)], sidx_v)
  pltpu.sync_copy(dstp_hbm.at[pl.ds(base, HSLAB)], didx_v)
  _zero_acc(zeros_hbm, shared, sid)
  plsc.subcore_barrier()

  pltpu.async_copy(table_hbm.at[sidx_v.at[0]], rows_v.at[0], gsem).wait()

  def step(j, _):
    b = lax.rem(j, 2)
    nb = 1 - b
    nxt = pltpu.make_async_copy(
        table_hbm.at[sidx_v.at[j + 1]], rows_v.at[nb], gsem)
    nxt.start()
    pltpu.async_copy(rows_v.at[b], shared.at[didx_v.at[j]], ssem,
                     add=True).wait()
    nxt.wait()
    return _

  lax.fori_loop(0, nch - 1, step, None)
  lb = lax.rem(nch - 1, 2)
  pltpu.async_copy(rows_v.at[lb], shared.at[didx_v.at[nch - 1]], ssem,
                   add=True).wait()
  plsc.subcore_barrier()
  _read_acc(shared, out_hbm, cid, sid)


def _agg_call(table, srcp, dstp, zeros128):
  return pl.kernel(
      _agg_body,
      out_type=jax.ShapeDtypeStruct((NC, NPAD, D), jnp.float32),
      mesh=_sc_mesh(),
      scratch_types=[
          pltpu.VMEM((HSLAB, CH), jnp.int32),
          pltpu.VMEM((HSLAB, CH), jnp.int32),
          pltpu.VMEM((2, CH, D), jnp.float32),
          pltpu.SemaphoreType.DMA,
          pltpu.SemaphoreType.DMA,
          pltpu.VMEM_SHARED((NACC, D), jnp.float32),
      ],
  )(table, srcp, dstp, zeros128)


RBLK = 1024


def _dinv_blk(d0, d1, blk_i):
  grow = blk_i * RBLK + lax.broadcasted_iota(jnp.int32, (RBLK, 1), 0)
  padc = jnp.where(grow < CH, float(NJUNK), 0.0)
  return lax.rsqrt(d0[:, :1] + d1[:, :1] + 1.0 - padc)


def _pad_mask(blk_i):
  grow = blk_i * RBLK + lax.broadcasted_iota(jnp.int32, (RBLK, 1), 0)
  return grow < N


def _prep_body(deg_ref, x_ref, xs_ref):
  i = pl.program_id(0)
  dinv = _dinv_blk(deg_ref[0], deg_ref[1], i)
  xs_ref[...] = jnp.where(_pad_mask(i), x_ref[...] * dinv, 0.0)


def _prep_call(deg, xp):
  grid = NPAD // RBLK
  return pl.pallas_call(
      _prep_body,
      grid=(grid,),
      in_specs=[
          pl.BlockSpec((NC, RBLK, D), lambda i: (0, i, 0)),
          pl.BlockSpec((RBLK, D), lambda i: (i, 0)),
      ],
      out_specs=pl.BlockSpec((RBLK, D), lambda i: (i, 0)),
      out_shape=jax.ShapeDtypeStruct((NPAD, D), jnp.float32),
  )(deg, xp)


def _layer_body(deg_ref, agg_ref, xs_ref, w_ref, b_ref, *out_refs,
                nout: int):
  i = pl.program_id(0)
  dinv = _dinv_blk(deg_ref[0], deg_ref[1], i)
  agg = agg_ref[0] + agg_ref[1] + agg_ref[2] + agg_ref[3]
  p = dinv * (agg + xs_ref[...])
  h = jnp.dot(p, w_ref[...], preferred_element_type=jnp.float32)
  h = jnp.maximum(h + b_ref[...], 0.0)
  h = jnp.where(_pad_mask(i), h * dinv, 0.0)
  fo = h.shape[-1] // nout
  for t in range(nout):
    out_refs[t][...] = h[:, t * fo:(t + 1) * fo]


def _layer_call(deg, agg, xs, w, b, nout: int):
  fi, fo = w.shape
  grid = NPAD // RBLK
  body = functools.partial(_layer_body, nout=nout)
  return pl.pallas_call(
      body,
      grid=(grid,),
      in_specs=[
          pl.BlockSpec((NC, RBLK, D), lambda i: (0, i, 0)),
          pl.BlockSpec((2 * NC, RBLK, fi), lambda i: (0, i, 0)),
          pl.BlockSpec((RBLK, fi), lambda i: (i, 0)),
          pl.BlockSpec((fi, fo), lambda i: (0, 0)),
          pl.BlockSpec((1, fo), lambda i: (0, 0)),
      ],
      out_specs=[pl.BlockSpec((RBLK, fo // nout), lambda i: (i, 0))
                 for _ in range(nout)],
      out_shape=[jax.ShapeDtypeStruct((NPAD, fo // nout), jnp.float32)
                 for _ in range(nout)],
  )(deg, agg, xs, w, b)


def _layer3_body(deg_ref, agglo_ref, agghi_ref, xslo_ref, xshi_ref,
                 w3a_ref, w3b_ref, b_ref, h3_ref):
  i = pl.program_id(0)
  dinv = _dinv_blk(deg_ref[0], deg_ref[1], i)
  alo = agglo_ref[0] + agglo_ref[1] + agglo_ref[2] + agglo_ref[3]
  ahi = agghi_ref[0] + agghi_ref[1] + agghi_ref[2] + agghi_ref[3]
  plo = dinv * (alo + xslo_ref[...])
  phi = dinv * (ahi + xshi_ref[...])
  h = (jnp.dot(plo, w3a_ref[...], preferred_element_type=jnp.float32)
       + jnp.dot(phi, w3b_ref[...], preferred_element_type=jnp.float32))
  h3_ref[...] = jnp.where(_pad_mask(i), jnp.maximum(h + b_ref[...], 0.0),
                          0.0)


def _layer3_call(deg, agglo, agghi, xslo, xshi, w3a, w3b, b3):
  grid = NPAD // RBLK
  return pl.pallas_call(
      _layer3_body,
      grid=(grid,),
      in_specs=[
          pl.BlockSpec((NC, RBLK, D), lambda i: (0, i, 0)),
          pl.BlockSpec((2 * NC, RBLK, D), lambda i: (0, i, 0)),
          pl.BlockSpec((2 * NC, RBLK, D), lambda i: (0, i, 0)),
          pl.BlockSpec((RBLK, D), lambda i: (i, 0)),
          pl.BlockSpec((RBLK, D), lambda i: (i, 0)),
          pl.BlockSpec((D, 4 * D), lambda i: (0, 0)),
          pl.BlockSpec((D, 4 * D), lambda i: (0, 0)),
          pl.BlockSpec((1, 4 * D), lambda i: (0, 0)),
      ],
      out_specs=pl.BlockSpec((RBLK, 4 * D), lambda i: (i, 0)),
      out_shape=jax.ShapeDtypeStruct((NPAD, 4 * D), jnp.float32),
  )(deg, agglo, agghi, xslo, xshi, w3a, w3b, b3)


def _tcpool_body(batch_ref, h3_ref, g_ref):
  i = pl.program_id(0)
  bblk = batch_ref[...]
  oh = jnp.where(
      lax.broadcasted_iota(jnp.int32, (B, RBLK), 0)
      == jnp.broadcast_to(bblk, (B, RBLK)), 1.0, 0.0).astype(jnp.float32)
  p = jnp.dot(oh, h3_ref[...], preferred_element_type=jnp.float32)

  @pl.when(i == 0)
  def _():
    g_ref[...] = p

  @pl.when(i > 0)
  def _():
    g_ref[...] += p


def _tcpool_call(batch2, h3):
  grid = NPAD // RBLK
  return pl.pallas_call(
      _tcpool_body,
      grid=(grid,),
      in_specs=[
          pl.BlockSpec((1, RBLK), lambda i: (0, i)),
          pl.BlockSpec((RBLK, 4 * D), lambda i: (i, 0)),
      ],
      out_specs=pl.BlockSpec((B, 4 * D), lambda i: (0, 0)),
      out_shape=jax.ShapeDtypeStruct((B, 4 * D), jnp.float32),
  )(batch2, h3)


def _conva_body(target_ref, wc2_ref, pn_ref):
  t3 = jnp.broadcast_to(target_ref[...].reshape(B, 1, L), (B, 26, L))
  t3 = t3.reshape(B * 26, L)
  cls = lax.broadcasted_iota(jnp.int32, (B * 26, L), 0) % 26
  oh = jnp.where(cls == t3, 1.0, 0.0).astype(jnp.float32)
  pn_ref[...] = jnp.dot(oh, wc2_ref[...], preferred_element_type=jnp.float32)


def _conva_call(target, wc2):
  return pl.pallas_call(
      _conva_body,
      out_shape=jax.ShapeDtypeStruct((B * 26, 8 * 32), jnp.float32),
  )(target, wc2)


def _convb_body(e7_ref, p4_ref, xtj_ref):
  xtj_ref[...] = jnp.dot(e7_ref[...], p4_ref[...],
                         preferred_element_type=jnp.float32)


def _convb_call(e7, p4):
  return pl.pallas_call(
      _convb_body,
      out_shape=jax.ShapeDtypeStruct((121, B * 32), jnp.float32),
  )(e7, p4)


def _head_body(gp_ref, xtflat_ref, wxt_ref, bxt_ref,
               wg1_ref, bg1_ref, wg2_ref, bg2_ref,
               wf1_ref, bf1_ref, wf2_ref, bf2_ref, wo_ref, bo_ref,
               out_ref):
  xt = jnp.dot(xtflat_ref[...], wxt_ref[...],
               preferred_element_type=jnp.float32) + bxt_ref[...]

  g = gp_ref[...]
  g = jnp.maximum(jnp.dot(g, wg1_ref[...],
                          preferred_element_type=jnp.float32)
                  + bg1_ref[...], 0.0)
  g = jnp.dot(g, wg2_ref[...], preferred_element_type=jnp.float32) \
      + bg2_ref[...]

  xc = jnp.concatenate([g, xt], axis=1)
  xc = jnp.maximum(jnp.dot(xc, wf1_ref[...],
                           preferred_element_type=jnp.float32)
                   + bf1_ref[...], 0.0)
  xc = jnp.maximum(jnp.dot(xc, wf2_ref[...],
                           preferred_element_type=jnp.float32)
                   + bf2_ref[...], 0.0)
  out_ref[...] = jnp.dot(xc, wo_ref[...],
                         preferred_element_type=jnp.float32) + bo_ref[...]


def _head_call(gp, xtflat, wxt, bxt, wg1, bg1, wg2, bg2,
               wf1, bf1, wf2, bf2, wo, bo):
  return pl.pallas_call(
      _head_body,
      out_shape=jax.ShapeDtypeStruct((B, 1), jnp.float32),
  )(gp, xtflat, wxt, bxt, wg1, bg1, wg2, bg2, wf1, bf1, wf2, bf2, wo, bo)


def kernel(x, edge_index, batch, target, W1, b1, W2, b2, W3, b3,
           Wg1, bg1, Wg2, bg2, emb, Wc, bc, Wxt, bxt,
           Wf1, bf1, Wf2, bf2, Wo, bo):
  f32 = jnp.float32

  def _junk(nrow, mod, off):
    j = jnp.arange(nrow * CH, dtype=jnp.int32).reshape(nrow, CH)
    return off + (j % mod)

  srcr = edge_index[0].reshape(NCHUNKS, CH)
  dstr = edge_index[1].reshape(NCHUNKS, CH)
  npadrow = NROWS_IDX - NCHUNKS
  srcp = jnp.concatenate([srcr, _junk(npadrow, NPAD - N, N)])
  dstp = jnp.concatenate([dstr, _junk(npadrow, CH, 0)])
  hpad = HROWS_IDX - HCHUNKS
  srcA = jnp.concatenate([srcr[:HCHUNKS], _junk(hpad, NPAD - N, N)])
  srcB = jnp.concatenate([srcr[HCHUNKS:], _junk(hpad, NPAD - N, N)])
  dstA = jnp.concatenate([dstr[:HCHUNKS], _junk(hpad, CH, 0)])
  dstB = jnp.concatenate([dstr[HCHUNKS:], _junk(hpad, CH, 0)])

  xp = jnp.pad(x, ((0, NPAD - N), (0, 0)))
  batch2 = jnp.concatenate(
      [batch, jnp.full((NPAD - N,), B, jnp.int32)]).reshape(1, NPAD)

  ones128 = jnp.ones((CH, D), f32)
  zeros128 = jnp.zeros((CH, D), f32)

  b1r = b1.reshape(1, -1)
  b2r = b2.reshape(1, -1)
  b3r = b3.reshape(1, -1)
  w3a, w3b = W3[:D], W3[D:]

  wc2 = Wc.transpose(1, 2, 0).reshape(L, 8 * 32)
  e7 = jnp.stack([emb[:, k:k + 121] for k in range(8)], axis=1).reshape(
      26 * 8, 121).T

  def _agg2(table):
    pa = _agg_call(table, srcA, dstA, zeros128)
    pb = _agg_call(table, srcB, dstB, zeros128)
    return jnp.concatenate([pa, pb])

  deg = _deg_call(dstp, ones128, zeros128)
  xs0 = _prep_call(deg, xp)
  a0 = _agg2(xs0)
  (xs1,) = _layer_call(deg, a0, xs0, W1, b1r, nout=1)
  a1 = _agg2(xs1)
  xs2lo, xs2hi = _layer_call(deg, a1, xs1, W2, b2r, nout=2)
  a2lo = _agg2(xs2lo)
  a2hi = _agg2(xs2hi)
  h3 = _layer3_call(deg, a2lo, a2hi, xs2lo, xs2hi, w3a, w3b, b3r)
  gp = _tcpool_call(batch2, h3)

  pn = _conva_call(target, wc2)
  p4 = pn.reshape(B, 26, 8, 32).transpose(1, 2, 0, 3).reshape(208, B * 32)
  xtj = _convb_call(e7, p4)
  xtflat = xtj.reshape(121, B, 32).transpose(1, 2, 0).reshape(B, 32 * 121)

  out = _head_call(gp, xtflat, Wxt, bxt.reshape(1, -1),
                   Wg1, bg1.reshape(1, -1), Wg2, bg2.reshape(1, -1),
                   Wf1, bf1.reshape(1, -1), Wf2, bf2.reshape(1, -1),
                   Wo, bo.reshape(1, -1))
  return out
